# Initial kernel scaffold; baseline (speedup 1.0000x reference)
import functools
import math

import jax
import jax.numpy as jnp
import numpy as np
from jax import lax
from jax.experimental import pallas as pl
from jax.experimental.pallas import tpu as pltpu
from jax.experimental.pallas import tpu_sc as plsc

NLIG = 2048
NPROT = 8192
NTOT = NLIG + NPROT
K = 48
HID = 128
DEPTH = 5
E = NLIG * K
CPAD = 16

_NFREQ = int(math.log(15.0 / (15.0 / 2000.0), 4)) + 1
_FREQS_NP = (2.0 * math.pi * (4.0 ** np.arange(_NFREQ)) / 15.0).astype(np.float32)

NT = 16
TN = NLIG // NT
TE = TN * K

NW = 32
PER_W = E // NW
CHUNK = 512
NCHUNK = PER_W // CHUNK


def _make_sc_gather(with_coor: bool):
    mesh = plsc.VectorSubcoreMesh(core_axis_name="c", subcore_axis_name="s")
    out_type = [jax.ShapeDtypeStruct((E, HID), jnp.float32)]
    scratch = [
        pltpu.VMEM((NCHUNK, CHUNK), jnp.int32),
        pltpu.VMEM((CHUNK, HID), jnp.float32),
        pltpu.SemaphoreType.DMA,
    ]
    if with_coor:
        out_type.append(jax.ShapeDtypeStruct((E, CPAD), jnp.float32))
        scratch.append(pltpu.VMEM((CHUNK, CPAD), jnp.float32))

    def body(*refs):
        if with_coor:
            (h_hbm, c_hbm, idx_hbm, ho_hbm, co_hbm, idx_v, hbuf, sem, cbuf) = refs
        else:
            (h_hbm, idx_hbm, ho_hbm, idx_v, hbuf, sem) = refs
        wid = lax.axis_index("s") * 2 + lax.axis_index("c")
        base = wid * PER_W
        for ch in range(NCHUNK):
            pltpu.sync_copy(idx_hbm.at[pl.ds(base + ch * CHUNK, CHUNK)],
                            idx_v.at[ch])
            pltpu.async_copy(h_hbm.at[idx_v.at[ch]], hbuf, sem).wait()
            pltpu.sync_copy(hbuf, ho_hbm.at[pl.ds(base + ch * CHUNK, CHUNK)])
            if with_coor:
                pltpu.async_copy(c_hbm.at[idx_v.at[ch]], cbuf, sem).wait()
                pltpu.sync_copy(cbuf, co_hbm.at[pl.ds(base + ch * CHUNK, CHUNK)])

    return functools.partial(pl.kernel, mesh=mesh, out_type=out_type,
                             scratch_types=scratch)(body)


_sc_gather_hc = _make_sc_gather(True)
_sc_gather_h = _make_sc_gather(False)


def _gather_hc(h_tab, c_tab, idx):
    return _sc_gather_hc(h_tab, c_tab, idx)


def _gather_h(h_tab, idx):
    return _sc_gather_h(h_tab, idx)[0]


def _silu(x):
    return x * jax.nn.sigmoid(x)


def _dot(a, b):
    return jnp.dot(a, b, preferred_element_type=jnp.float32)


def _expand_rows(x, reps, cols):
    n = x.shape[0]
    return jnp.broadcast_to(x[:, None, :], (n, reps, cols)).reshape(n * reps, cols)


def _edge_geom(cc, xr):
    xe = _expand_rows(xr, K, CPAD)
    diff = xe - cc
    d2 = jnp.sum(diff * diff, axis=1, keepdims=True)
    norm = jnp.sqrt(d2 + 1e-8)
    cdiff = diff / (norm + 1.0)
    return d2, cdiff, norm


def _sin_cos(norm):
    freqs = jnp.asarray(_FREQS_NP).reshape(1, _NFREQ)
    e = norm * freqs
    return jnp.sin(e), jnp.cos(e)


def _prep_body(xf, t, ele, aa, bb, temb, eemb, aemb, bemb,
               wx, wt, we, wa, wb, bi, out):
    tt = _dot(temb[...], wt[...])
    te = _dot(eemb[...], we[...])
    ta = _dot(aemb[...], wa[...])
    tb = _dot(bemb[...], wb[...])
    oh_t = (lax.broadcasted_iota(jnp.int32, (NLIG, 1000), 1) == t[...]).astype(jnp.float32)
    hl = _dot(xf[...], wx[...]) + _dot(oh_t, tt) + bi[...]
    oh_e = (lax.broadcasted_iota(jnp.int32, (NPROT, 5), 1) == ele[...]).astype(jnp.float32)
    oh_a = (lax.broadcasted_iota(jnp.int32, (NPROT, 20), 1) == aa[...]).astype(jnp.float32)
    oh_b = (lax.broadcasted_iota(jnp.int32, (NPROT, 2), 1) == bb[...]).astype(jnp.float32)
    hp = _dot(oh_e, te) + _dot(oh_a, ta) + _dot(oh_b, tb) + bi[...]
    out[0:NLIG, :] = hl
    out[NLIG:NTOT, :] = hp


def _prep(xf, t, ele, aa, bb, temb, eemb, aemb, bemb, wx, wt, we, wa, wb, bi):
    full = lambda s: pl.BlockSpec(s, lambda: (0,) * len(s))
    return pl.pallas_call(
        _prep_body,
        grid=(),
        in_specs=[full((NLIG, 32)), full((NLIG, 1)), full((NPROT, 1)),
                  full((NPROT, 1)), full((NPROT, 1)), full((1000, 8)),
                  full((5, 16)), full((20, 16)), full((2, 8)),
                  full((32, HID)), full((8, HID)), full((16, HID)),
                  full((16, HID)), full((8, HID)), full((1, HID))],
        out_specs=full((NTOT, HID)),
        out_shape=jax.ShapeDtypeStruct((NTOT, HID), jnp.float32),
    )(xf, t, ele, aa, bb, temb, eemb, aemb, bemb, wx, wt, we, wa, wb, bi)


_D2T = 256


def _d2_body(lig, ct, out):
    lt = lig[...]
    c = ct[...]
    cross = _dot(lt, c)
    lig2 = jnp.sum(lt * lt, axis=1, keepdims=True)
    all2 = jnp.sum(c * c, axis=0, keepdims=True)
    d2 = lig2 + all2 - 2.0 * cross
    i = pl.program_id(0)
    rowid = lax.broadcasted_iota(jnp.int32, (_D2T, NTOT), 0) + i * _D2T
    colid = lax.broadcasted_iota(jnp.int32, (_D2T, NTOT), 1)
    out[...] = jnp.where(rowid == colid, jnp.inf, d2)


def _d2_matrix(lig4, coorsT4):
    return pl.pallas_call(
        _d2_body,
        grid=(NLIG // _D2T,),
        in_specs=[pl.BlockSpec((_D2T, 4), lambda i: (i, 0)),
                  pl.BlockSpec((4, NTOT), lambda i: (0, 0))],
        out_specs=pl.BlockSpec((_D2T, NTOT), lambda i: (i, 0)),
        out_shape=jax.ShapeDtypeStruct((NLIG, NTOT), jnp.float32),
    )(lig4, coorsT4)


def _norm0_body(cc, xr, out):
    d2, _, norm = _edge_geom(cc[...], xr[...])
    out[...] = norm


def _norm0(ccol, xlig):
    return pl.pallas_call(
        _norm0_body,
        grid=(NT,),
        in_specs=[pl.BlockSpec((TE, CPAD), lambda i: (i, 0)),
                  pl.BlockSpec((TN, CPAD), lambda i: (i, 0))],
        out_specs=pl.BlockSpec((TE, 1), lambda i: (i, 0)),
        out_shape=jax.ShapeDtypeStruct((E, 1), jnp.float32),
    )(ccol, xlig)


def _edge_pre(hcol, cc, xr, n0, hrow_part, w1hc, w1ss, w1sc, w1s0s, w1s0c):
    d2, cdiff, norm = _edge_geom(cc, xr)
    sb_s, sb_c = _sin_cos(norm)
    s0_s, s0_c = _sin_cos(n0)
    z = _dot(hcol, w1hc)
    z = z + _dot(sb_s, w1ss) + _dot(sb_c, w1sc)
    z = z + _dot(s0_s, w1s0s) + _dot(s0_c, w1s0c)
    z = z + _expand_rows(hrow_part, K, HID)
    return z, cdiff


def _gcl_body(hlig, hcol, cc, xr, n0,
              w1hr, w1hc, w1ss, w1sc, w1s0s, w1s0c, b1,
              w2, b2, wa, ba, wn1h, wn1a, bn1, wn2, bn2, out):
    h_t = hlig[...]
    hrow_part = _dot(h_t, w1hr[...]) + b1[...]
    z, _ = _edge_pre(hcol[...], cc[...], xr[...], n0[...], hrow_part,
                     w1hc[...], w1ss[...], w1sc[...], w1s0s[...], w1s0c[...])
    m1 = _silu(z)
    m2 = _silu(_dot(m1, w2[...]) + b2[...])
    g = jax.nn.sigmoid(_dot(m2, wa[...]) + ba[...])
    e = m2 * g
    agg = jnp.sum(e.reshape(TN, K, HID), axis=1) / 5.0
    u = _silu(_dot(h_t, wn1h[...]) + _dot(agg, wn1a[...]) + bn1[...])
    out[...] = h_t + _dot(u, wn2[...]) + bn2[...]


def _gcl(h_full, hcol, ccol, xlig, n0, w):
    full = lambda s: pl.BlockSpec(s, lambda i: (0,) * len(s))
    return pl.pallas_call(
        _gcl_body,
        grid=(NT,),
        in_specs=[pl.BlockSpec((TN, HID), lambda i: (i, 0)),
                  pl.BlockSpec((TE, HID), lambda i: (i, 0)),
                  pl.BlockSpec((TE, CPAD), lambda i: (i, 0)),
                  pl.BlockSpec((TN, CPAD), lambda i: (i, 0)),
                  pl.BlockSpec((TE, 1), lambda i: (i, 0)),
                  full((HID, HID)), full((HID, HID)),
                  full((_NFREQ, HID)), full((_NFREQ, HID)),
                  full((_NFREQ, HID)), full((_NFREQ, HID)), full((1, HID)),
                  full((HID, HID)), full((1, HID)),
                  full((HID, 1)), full((1, 1)),
                  full((HID, HID)), full((HID, HID)), full((1, HID)),
                  full((HID, HID)), full((1, HID))],
        out_specs=pl.BlockSpec((TN, HID), lambda i: (i, 0)),
        out_shape=jax.ShapeDtypeStruct((NLIG, HID), jnp.float32),
    )(h_full, hcol, ccol, xlig, n0, *w)


_PT = 512


def _prot_body(h, wn1h, bn1, wn2, bn2, out):
    h_t = h[...]
    u = _silu(_dot(h_t, wn1h[...]) + bn1[...])
    out[...] = h_t + _dot(u, wn2[...]) + bn2[...]


def _prot(h_full, wn1h, bn1, wn2, bn2):
    full = lambda s: pl.BlockSpec(s, lambda i: (0,) * len(s))
    return pl.pallas_call(
        _prot_body,
        grid=(NPROT // _PT,),
        in_specs=[pl.BlockSpec((_PT, HID), lambda i: (i + NLIG // _PT, 0)),
                  full((HID, HID)), full((1, HID)),
                  full((HID, HID)), full((1, HID))],
        out_specs=pl.BlockSpec((_PT, HID), lambda i: (i, 0)),
        out_shape=jax.ShapeDtypeStruct((NPROT, HID), jnp.float32),
    )(h_full, wn1h, bn1, wn2, bn2)


def _coord_body(hlig, hcol, cc, xr, n0,
                c1hr, c1hc, c1ss, c1sc, c1s0s, c1s0c, bc1,
                c2w, bc2, c3w, out):
    h_t = hlig[...]
    hrow_part = _dot(h_t, c1hr[...]) + bc1[...]
    z, cdiff = _edge_pre(hcol[...], cc[...], xr[...], n0[...], hrow_part,
                         c1hc[...], c1ss[...], c1sc[...], c1s0s[...], c1s0c[...])
    p1 = _silu(z)
    p2 = _silu(_dot(p1, c2w[...]) + bc2[...])
    phi = _dot(p2, c3w[...])
    trans = cdiff * phi
    tagg = jnp.sum(trans.reshape(TN, K, CPAD), axis=1) / 5.0
    out[...] = xr[...] + tagg


def _coord(h_full, hcol, ccol, xlig, n0, w):
    full = lambda s: pl.BlockSpec(s, lambda i: (0,) * len(s))
    return pl.pallas_call(
        _coord_body,
        grid=(NT,),
        in_specs=[pl.BlockSpec((TN, HID), lambda i: (i, 0)),
                  pl.BlockSpec((TE, HID), lambda i: (i, 0)),
                  pl.BlockSpec((TE, CPAD), lambda i: (i, 0)),
                  pl.BlockSpec((TN, CPAD), lambda i: (i, 0)),
                  pl.BlockSpec((TE, 1), lambda i: (i, 0)),
                  full((HID, HID)), full((HID, HID)),
                  full((_NFREQ, HID)), full((_NFREQ, HID)),
                  full((_NFREQ, HID)), full((_NFREQ, HID)), full((1, HID)),
                  full((HID, HID)), full((1, HID)), full((HID, 1))],
        out_specs=pl.BlockSpec((TN, CPAD), lambda i: (i, 0)),
        out_shape=jax.ShapeDtypeStruct((NLIG, CPAD), jnp.float32),
    )(h_full, hcol, ccol, xlig, n0, *w)


def _final_body(h, x, wo, bo, posw, hout, xout):
    hout[...] = _dot(h[...], wo[...]) + bo[...]
    xout[...] = x[...] * posw[...]


def _final(h_full, xlig, wo, bo, posw):
    full = lambda s: pl.BlockSpec(s, lambda: (0,) * len(s))
    return pl.pallas_call(
        _final_body,
        grid=(),
        in_specs=[full((NLIG, HID)), full((NLIG, CPAD)),
                  full((HID, 32)), full((1, 32)), full((1, 1))],
        out_specs=[full((NLIG, 32)), full((NLIG, CPAD))],
        out_shape=[jax.ShapeDtypeStruct((NLIG, 32), jnp.float32),
                   jax.ShapeDtypeStruct((NLIG, CPAD), jnp.float32)],
    )(h_full, xlig, wo, bo, posw)


def _row(b):
    return b.reshape(1, -1)


def _split_edge_w(w):
    return (w[0:128], w[128:256], w[256:262], w[262:268], w[268:274], w[274:280])


def _gcl_weights(g):
    w1 = _split_edge_w(g["edge1"]["w"])
    return (w1[0], w1[1], w1[2], w1[3], w1[4], w1[5], _row(g["edge1"]["b"]),
            g["edge2"]["w"], _row(g["edge2"]["b"]),
            g["att"]["w"], _row(g["att"]["b"]),
            g["node1"]["w"][0:128], g["node1"]["w"][128:256], _row(g["node1"]["b"]),
            g["node2"]["w"], _row(g["node2"]["b"]))


def _equiv_weights(q):
    c1 = _split_edge_w(q["c1"]["w"])
    return (c1[0], c1[1], c1[2], c1[3], c1[4], c1[5], _row(q["c1"]["b"]),
            q["c2"]["w"], _row(q["c2"]["b"]), q["c3"]["w"])


def kernel(protein_positions, protein_ele, protein_amino_acid,
           protein_is_backbone, Xt_pos, Xt_features, t, params):
    eg = params["egnn"]
    coors = jnp.concatenate([Xt_pos, protein_positions], axis=0)
    cpad = jnp.zeros((NTOT, CPAD), jnp.float32).at[:, :3].set(coors)
    x_lig = cpad[:NLIG]
    c_prot = cpad[NLIG:]

    d2 = _d2_matrix(cpad[:NLIG, :4], cpad[:, :4].T)
    col = lax.top_k(-d2, K)[1].reshape(E).astype(jnp.int32)

    wi = eg["emb_in"]["w"]
    h_full = _prep(Xt_features, t.reshape(NLIG, 1).astype(jnp.int32),
                   protein_ele.reshape(NPROT, 1).astype(jnp.int32),
                   protein_amino_acid.reshape(NPROT, 1).astype(jnp.int32),
                   protein_is_backbone.reshape(NPROT, 1).astype(jnp.int32),
                   params["time_emb"], params["prot_ele_emb"],
                   params["prot_aa_emb"], params["prot_bb_emb"],
                   wi[0:32], wi[32:40], wi[0:16], wi[16:32], wi[32:40],
                   _row(eg["emb_in"]["b"]))

    n0 = None
    for b in range(DEPTH):
        blk = eg["blocks"][b]
        hcol, ccol = _gather_hc(h_full, cpad, col)
        if b == 0:
            n0 = _norm0(ccol, x_lig)
        gw = _gcl_weights(blk["gcl"])
        h_lig = _gcl(h_full, hcol, ccol, x_lig, n0, gw)
        h_prot = _prot(h_full, gw[11], gw[13], gw[14], gw[15])
        h_full = jnp.concatenate([h_lig, h_prot], axis=0)
        hcol2 = _gather_h(h_full, col)
        x_lig = _coord(h_full, hcol2, ccol, x_lig, n0, _equiv_weights(blk["equiv"]))
        cpad = jnp.concatenate([x_lig, c_prot], axis=0)

    h_out, x_out = _final(h_full, x_lig, eg["emb_out"]["w"],
                          _row(eg["emb_out"]["b"]), params["pos_w"])
    return x_out[:, :3], h_out


# speedup vs baseline: 1.7889x; 1.7889x over previous
import functools
import math

import jax
import jax.numpy as jnp
import numpy as np
from jax import lax
from jax.experimental import pallas as pl
from jax.experimental.pallas import tpu as pltpu
from jax.experimental.pallas import tpu_sc as plsc

NLIG = 2048
NPROT = 8192
NTOT = NLIG + NPROT
K = 48
HID = 128
DEPTH = 5
E = NLIG * K
CPAD = 128

_NFREQ = int(math.log(15.0 / (15.0 / 2000.0), 4)) + 1
_FREQS_NP = (2.0 * math.pi * (4.0 ** np.arange(_NFREQ)) / 15.0).astype(np.float32)

NT = 16
TN = NLIG // NT
TE = TN * K

NW = 32
PER_W = E // NW
CHUNK = 128
NCHUNK = PER_W // CHUNK


@functools.lru_cache(maxsize=None)
def _make_sc_gather(with_coor: bool):
    mesh = plsc.VectorSubcoreMesh(core_axis_name="c", subcore_axis_name="s")
    out_type = [jax.ShapeDtypeStruct((E, HID), jnp.float32)]
    scratch = [
        pltpu.VMEM((PER_W,), jnp.int32),
        pltpu.VMEM((CHUNK, HID), jnp.float32),
        pltpu.SemaphoreType.DMA,
    ]
    if with_coor:
        out_type.append(jax.ShapeDtypeStruct((E, CPAD), jnp.float32))
        scratch.append(pltpu.VMEM((CHUNK, CPAD), jnp.float32))

    def body(*refs):
        if with_coor:
            (h_hbm, c_hbm, idx_hbm, ho_hbm, co_hbm, idx_v, hbuf, sem, cbuf) = refs
        else:
            (h_hbm, idx_hbm, ho_hbm, idx_v, hbuf, sem) = refs
        wid = lax.axis_index("s") * 2 + lax.axis_index("c")
        base = wid * PER_W
        pltpu.sync_copy(idx_hbm.at[pl.ds(base, PER_W)], idx_v)
        for ch in range(NCHUNK):
            idx_c = idx_v.at[pl.ds(ch * CHUNK, CHUNK)]
            pltpu.async_copy(h_hbm.at[idx_c], hbuf, sem).wait()
            pltpu.sync_copy(hbuf, ho_hbm.at[pl.ds(base + ch * CHUNK, CHUNK)])
            if with_coor:
                pltpu.async_copy(c_hbm.at[idx_c], cbuf, sem).wait()
                pltpu.sync_copy(cbuf, co_hbm.at[pl.ds(base + ch * CHUNK, CHUNK)])

    return functools.partial(pl.kernel, mesh=mesh, out_type=out_type,
                             scratch_types=scratch)(body)


def _gather_hc(h_tab, c_tab, idx):
    return _make_sc_gather(True)(h_tab, c_tab, idx)


def _gather_h(h_tab, idx):
    return _make_sc_gather(False)(h_tab, idx)[0]


def _silu(x):
    return x * jax.nn.sigmoid(x)


def _dot(a, b):
    return jnp.dot(a, b, preferred_element_type=jnp.float32)


def _expand_rows(x, reps, cols):
    n = x.shape[0]
    return jnp.broadcast_to(x[:, None, :], (n, reps, cols)).reshape(n * reps, cols)


def _edge_geom(cc, xr):
    xe = _expand_rows(xr, K, CPAD)
    diff = xe - cc
    d2 = jnp.sum(diff * diff, axis=1, keepdims=True)
    norm = jnp.sqrt(d2 + 1e-8)
    cdiff = diff / (norm + 1.0)
    return d2, cdiff, norm


def _sin_cos(norm):
    lane = lax.broadcasted_iota(jnp.int32, (1, _NFREQ), 1)
    freqs = jnp.zeros((1, _NFREQ), jnp.float32)
    for i, f in enumerate(_FREQS_NP):
        freqs = jnp.where(lane == i, float(f), freqs)
    e = norm * freqs
    return jnp.sin(e), jnp.cos(e)


def _prep_body(xf, t, ele, aa, bb, temb, eemb, aemb, bemb,
               wx, wt, we, wa, wb, bi, out):
    tt = _dot(temb[...], wt[...])
    te = _dot(eemb[...], we[...])
    ta = _dot(aemb[...], wa[...])
    tb = _dot(bemb[...], wb[...])
    oh_t = (lax.broadcasted_iota(jnp.int32, (NLIG, 1000), 1) == t[...]).astype(jnp.float32)
    hl = _dot(xf[...], wx[...]) + _dot(oh_t, tt) + bi[...]
    oh_e = (lax.broadcasted_iota(jnp.int32, (NPROT, 5), 1) == ele[...]).astype(jnp.float32)
    oh_a = (lax.broadcasted_iota(jnp.int32, (NPROT, 20), 1) == aa[...]).astype(jnp.float32)
    oh_b = (lax.broadcasted_iota(jnp.int32, (NPROT, 2), 1) == bb[...]).astype(jnp.float32)
    hp = _dot(oh_e, te) + _dot(oh_a, ta) + _dot(oh_b, tb) + bi[...]
    out[0:NLIG, :] = hl
    out[NLIG:NTOT, :] = hp


def _prep(xf, t, ele, aa, bb, temb, eemb, aemb, bemb, wx, wt, we, wa, wb, bi):
    full = lambda s: pl.BlockSpec(s, lambda i: (0,) * len(s))
    return pl.pallas_call(
        _prep_body,
        grid=(1,),
        in_specs=[full((NLIG, 32)), full((NLIG, 1)), full((NPROT, 1)),
                  full((NPROT, 1)), full((NPROT, 1)), full((1000, 8)),
                  full((5, 16)), full((20, 16)), full((2, 8)),
                  full((32, HID)), full((8, HID)), full((16, HID)),
                  full((16, HID)), full((8, HID)), full((1, HID))],
        out_specs=full((NTOT, HID)),
        out_shape=jax.ShapeDtypeStruct((NTOT, HID), jnp.float32),
    )(xf, t, ele, aa, bb, temb, eemb, aemb, bemb, wx, wt, we, wa, wb, bi)


_D2T = 256


def _d2_body(lig, ct, out):
    lt = lig[...]
    c = ct[...]
    cross = _dot(lt, c)
    lig2 = jnp.sum(lt * lt, axis=1, keepdims=True)
    all2 = jnp.sum(c * c, axis=0, keepdims=True)
    d2 = lig2 + all2 - 2.0 * cross
    i = pl.program_id(0)
    rowid = lax.broadcasted_iota(jnp.int32, (_D2T, NTOT), 0) + i * _D2T
    colid = lax.broadcasted_iota(jnp.int32, (_D2T, NTOT), 1)
    out[...] = jnp.where(rowid == colid, jnp.inf, d2)


def _d2_matrix(lig4, coorsT4):
    return pl.pallas_call(
        _d2_body,
        grid=(NLIG // _D2T,),
        in_specs=[pl.BlockSpec((_D2T, 4), lambda i: (i, 0)),
                  pl.BlockSpec((4, NTOT), lambda i: (0, 0))],
        out_specs=pl.BlockSpec((_D2T, NTOT), lambda i: (i, 0)),
        out_shape=jax.ShapeDtypeStruct((NLIG, NTOT), jnp.float32),
    )(lig4, coorsT4)


def _norm0_body(cc, xr, out):
    d2, _, norm = _edge_geom(cc[...], xr[...])
    out[...] = norm


def _norm0(ccol, xlig):
    return pl.pallas_call(
        _norm0_body,
        grid=(NT,),
        in_specs=[pl.BlockSpec((TE, CPAD), lambda i: (i, 0)),
                  pl.BlockSpec((TN, CPAD), lambda i: (i, 0))],
        out_specs=pl.BlockSpec((TE, 1), lambda i: (i, 0)),
        out_shape=jax.ShapeDtypeStruct((E, 1), jnp.float32),
    )(ccol, xlig)


def _edge_pre(hcol, cc, xr, n0, hrow_part, w1hc, w1ss, w1sc, w1s0s, w1s0c):
    d2, cdiff, norm = _edge_geom(cc, xr)
    sb_s, sb_c = _sin_cos(norm)
    s0_s, s0_c = _sin_cos(n0)
    z = _dot(hcol, w1hc)
    z = z + _dot(sb_s, w1ss) + _dot(sb_c, w1sc)
    z = z + _dot(s0_s, w1s0s) + _dot(s0_c, w1s0c)
    z = z + _expand_rows(hrow_part, K, HID)
    return z, cdiff


def _gcl_body(hlig, hcol, cc, xr, n0,
              w1hr, w1hc, w1ss, w1sc, w1s0s, w1s0c, b1,
              w2, b2, wa, ba, wn1h, wn1a, bn1, wn2, bn2, out):
    h_t = hlig[...]
    hrow_part = _dot(h_t, w1hr[...]) + b1[...]
    z, _ = _edge_pre(hcol[...], cc[...], xr[...], n0[...], hrow_part,
                     w1hc[...], w1ss[...], w1sc[...], w1s0s[...], w1s0c[...])
    m1 = _silu(z)
    m2 = _silu(_dot(m1, w2[...]) + b2[...])
    g = jax.nn.sigmoid(_dot(m2, wa[...]) + ba[...])
    e = m2 * g
    agg = jnp.sum(e.reshape(TN, K, HID), axis=1) / 5.0
    u = _silu(_dot(h_t, wn1h[...]) + _dot(agg, wn1a[...]) + bn1[...])
    out[...] = h_t + _dot(u, wn2[...]) + bn2[...]


def _gcl(h_full, hcol, ccol, xlig, n0, w):
    full = lambda s: pl.BlockSpec(s, lambda i: (0,) * len(s))
    return pl.pallas_call(
        _gcl_body,
        grid=(NT,),
        in_specs=[pl.BlockSpec((TN, HID), lambda i: (i, 0)),
                  pl.BlockSpec((TE, HID), lambda i: (i, 0)),
                  pl.BlockSpec((TE, CPAD), lambda i: (i, 0)),
                  pl.BlockSpec((TN, CPAD), lambda i: (i, 0)),
                  pl.BlockSpec((TE, 1), lambda i: (i, 0)),
                  full((HID, HID)), full((HID, HID)),
                  full((_NFREQ, HID)), full((_NFREQ, HID)),
                  full((_NFREQ, HID)), full((_NFREQ, HID)), full((1, HID)),
                  full((HID, HID)), full((1, HID)),
                  full((HID, 1)), full((1, 1)),
                  full((HID, HID)), full((HID, HID)), full((1, HID)),
                  full((HID, HID)), full((1, HID))],
        out_specs=pl.BlockSpec((TN, HID), lambda i: (i, 0)),
        out_shape=jax.ShapeDtypeStruct((NLIG, HID), jnp.float32),
    )(h_full, hcol, ccol, xlig, n0, *w)


_PT = 512


def _prot_body(h, wn1h, bn1, wn2, bn2, out):
    h_t = h[...]
    u = _silu(_dot(h_t, wn1h[...]) + bn1[...])
    out[...] = h_t + _dot(u, wn2[...]) + bn2[...]


def _prot(h_full, wn1h, bn1, wn2, bn2):
    full = lambda s: pl.BlockSpec(s, lambda i: (0,) * len(s))
    return pl.pallas_call(
        _prot_body,
        grid=(NPROT // _PT,),
        in_specs=[pl.BlockSpec((_PT, HID), lambda i: (i + NLIG // _PT, 0)),
                  full((HID, HID)), full((1, HID)),
                  full((HID, HID)), full((1, HID))],
        out_specs=pl.BlockSpec((_PT, HID), lambda i: (i, 0)),
        out_shape=jax.ShapeDtypeStruct((NPROT, HID), jnp.float32),
    )(h_full, wn1h, bn1, wn2, bn2)


def _coord_body(hlig, hcol, cc, xr, n0,
                c1hr, c1hc, c1ss, c1sc, c1s0s, c1s0c, bc1,
                c2w, bc2, c3w, out):
    h_t = hlig[...]
    hrow_part = _dot(h_t, c1hr[...]) + bc1[...]
    z, cdiff = _edge_pre(hcol[...], cc[...], xr[...], n0[...], hrow_part,
                         c1hc[...], c1ss[...], c1sc[...], c1s0s[...], c1s0c[...])
    p1 = _silu(z)
    p2 = _silu(_dot(p1, c2w[...]) + bc2[...])
    phi = _dot(p2, c3w[...])
    trans = cdiff * phi
    tagg = jnp.sum(trans.reshape(TN, K, CPAD), axis=1) / 5.0
    out[...] = xr[...] + tagg


def _coord(h_full, hcol, ccol, xlig, n0, w):
    full = lambda s: pl.BlockSpec(s, lambda i: (0,) * len(s))
    return pl.pallas_call(
        _coord_body,
        grid=(NT,),
        in_specs=[pl.BlockSpec((TN, HID), lambda i: (i, 0)),
                  pl.BlockSpec((TE, HID), lambda i: (i, 0)),
                  pl.BlockSpec((TE, CPAD), lambda i: (i, 0)),
                  pl.BlockSpec((TN, CPAD), lambda i: (i, 0)),
                  pl.BlockSpec((TE, 1), lambda i: (i, 0)),
                  full((HID, HID)), full((HID, HID)),
                  full((_NFREQ, HID)), full((_NFREQ, HID)),
                  full((_NFREQ, HID)), full((_NFREQ, HID)), full((1, HID)),
                  full((HID, HID)), full((1, HID)), full((HID, 1))],
        out_specs=pl.BlockSpec((TN, CPAD), lambda i: (i, 0)),
        out_shape=jax.ShapeDtypeStruct((NLIG, CPAD), jnp.float32),
    )(h_full, hcol, ccol, xlig, n0, *w)


def _final_body(h, x, wo, bo, posw, hout, xout):
    hout[...] = _dot(h[...], wo[...]) + bo[...]
    xout[...] = x[...] * posw[...]


def _final(h_full, xlig, wo, bo, posw):
    full = lambda s: pl.BlockSpec(s, lambda i: (0,) * len(s))
    return pl.pallas_call(
        _final_body,
        grid=(1,),
        in_specs=[full((NLIG, HID)), full((NLIG, CPAD)),
                  full((HID, 32)), full((1, 32)), full((1, 1))],
        out_specs=[full((NLIG, 32)), full((NLIG, CPAD))],
        out_shape=[jax.ShapeDtypeStruct((NLIG, 32), jnp.float32),
                   jax.ShapeDtypeStruct((NLIG, CPAD), jnp.float32)],
    )(h_full, xlig, wo, bo, posw)


def _row(b):
    return b.reshape(1, -1)


def _split_edge_w(w):
    return (w[0:128], w[128:256], w[256:262], w[262:268], w[268:274], w[274:280])


def _gcl_weights(g):
    w1 = _split_edge_w(g["edge1"]["w"])
    return (w1[0], w1[1], w1[2], w1[3], w1[4], w1[5], _row(g["edge1"]["b"]),
            g["edge2"]["w"], _row(g["edge2"]["b"]),
            g["att"]["w"], _row(g["att"]["b"]),
            g["node1"]["w"][0:128], g["node1"]["w"][128:256], _row(g["node1"]["b"]),
            g["node2"]["w"], _row(g["node2"]["b"]))


def _equiv_weights(q):
    c1 = _split_edge_w(q["c1"]["w"])
    return (c1[0], c1[1], c1[2], c1[3], c1[4], c1[5], _row(q["c1"]["b"]),
            q["c2"]["w"], _row(q["c2"]["b"]), q["c3"]["w"])


def kernel(protein_positions, protein_ele, protein_amino_acid,
           protein_is_backbone, Xt_pos, Xt_features, t, params):
    eg = params["egnn"]
    coors = jnp.concatenate([Xt_pos, protein_positions], axis=0)
    cpad = jnp.zeros((NTOT, CPAD), jnp.float32).at[:, :3].set(coors)
    x_lig = cpad[:NLIG]
    c_prot = cpad[NLIG:]

    d2 = _d2_matrix(cpad[:NLIG, :4], cpad[:, :4].T)
    col = lax.top_k(-d2, K)[1].reshape(E).astype(jnp.int32)

    wi = eg["emb_in"]["w"]
    h_full = _prep(Xt_features, t.reshape(NLIG, 1).astype(jnp.int32),
                   protein_ele.reshape(NPROT, 1).astype(jnp.int32),
                   protein_amino_acid.reshape(NPROT, 1).astype(jnp.int32),
                   protein_is_backbone.reshape(NPROT, 1).astype(jnp.int32),
                   params["time_emb"], params["prot_ele_emb"],
                   params["prot_aa_emb"], params["prot_bb_emb"],
                   wi[0:32], wi[32:40], wi[0:16], wi[16:32], wi[32:40],
                   _row(eg["emb_in"]["b"]))

    n0 = None
    for b in range(DEPTH):
        blk = eg["blocks"][b]
        hcol, ccol = _gather_hc(h_full, cpad, col)
        if b == 0:
            n0 = _norm0(ccol, x_lig)
        gw = _gcl_weights(blk["gcl"])
        h_lig = _gcl(h_full, hcol, ccol, x_lig, n0, gw)
        h_prot = _prot(h_full, gw[11], gw[13], gw[14], gw[15])
        h_full = jnp.concatenate([h_lig, h_prot], axis=0)
        hcol2 = _gather_h(h_full, col)
        x_lig = _coord(h_full, hcol2, ccol, x_lig, n0, _equiv_weights(blk["equiv"]))
        cpad = jnp.concatenate([x_lig, c_prot], axis=0)

    h_out, x_out = _final(h_full, x_lig, eg["emb_out"]["w"],
                          _row(eg["emb_out"]["b"]), params["pos_w"])
    return x_out[:, :3], h_out
